# baseline (speedup 1.0000x reference)
import functools

import numpy as np
import jax
import jax.numpy as jnp
from jax import lax
from jax.experimental import pallas as pl

_ROPE_THETA = 10000.0


def _bf(a):
    return a.astype(jnp.bfloat16)


def _proj(a, b, cs=512):
    ab = _bf(a)
    bb = _bf(b)
    kdim = a.shape[1]
    chunks = [jnp.dot(ab[:, c:c + cs], bb[c:c + cs, :],
                      preferred_element_type=jnp.float32)
              for c in range(0, kdim, cs)]
    s = chunks[0]
    comp = jnp.zeros_like(s)
    for c in chunks[1:]:
        t = s + c
        big = jnp.abs(s) >= jnp.abs(c)
        corr = jnp.where(big, (s - t) + c, (c - t) + s)
        comp = comp + corr
        s = t
    return s + comp


def _rope_tables(seqlen, head_dim):
    half = head_dim // 2
    inv_freq = 1.0 / (_ROPE_THETA ** (np.arange(half, dtype=np.float64) / half))
    t = np.arange(seqlen, dtype=np.float64)
    ang = np.outer(t, inv_freq)
    return np.cos(ang).astype(np.float32), np.sin(ang).astype(np.float32)


def _k_body(x_ref, wk_ref, cos_ref, sin_ref, out_ref):
    xk = _proj(x_ref[0], wk_ref[...])
    half = xk.shape[1] // 2
    ka = xk[:, :half]
    kb = xk[:, half:]
    c = cos_ref[...]
    s = sin_ref[...]
    out_ref[0] = jnp.concatenate([ka * c - kb * s, ka * s + kb * c], axis=1)


def _bitonic_desc(v, ix, n):
    li = lax.broadcasted_iota(jnp.int32, (1, n), 1)
    ksz = 2
    while ksz <= n:
        j = ksz // 2
        while j >= 1:
            mlow = (li & j) == 0
            dmask = (li & ksz) == 0
            take_max = jnp.logical_not(jnp.logical_xor(mlow, dmask))
            vp = jnp.where(mlow, jnp.roll(v, -j, axis=1), jnp.roll(v, j, axis=1))
            ip = jnp.where(mlow, jnp.roll(ix, -j, axis=1), jnp.roll(ix, j, axis=1))
            stay = (take_max & (v >= vp)) | (jnp.logical_not(take_max) & (v <= vp))
            v = jnp.where(stay, v, vp)
            ix = jnp.where(stay, ix, ip)
            j //= 2
        ksz *= 2
    return v, ix


def _score_body(x_ref, wq_ref, ww_ref, k_ref, cos_ref, sin_ref,
                vals_ref, idx_ref, *, H, DH, blk, topk, S):
    i = pl.program_id(1)
    x = x_ref[0]
    q = _proj(x, wq_ref[...])
    w = _proj(x, ww_ref[...]) * (DH ** -0.5)
    c = cos_ref[...]
    s = sin_ref[...]
    kr = k_ref[0]
    half = DH // 2
    scores = jnp.zeros((blk, S), jnp.float32)
    for h in range(H):
        qa = q[:, h * DH:h * DH + half]
        qb = q[:, h * DH + half:(h + 1) * DH]
        qh = jnp.concatenate([qa * c - qb * s, qa * s + qb * c], axis=1)
        lg = lax.dot_general(_bf(qh), _bf(kr), (((1,), (1,)), ((), ())),
                             preferred_element_type=jnp.float32)
        scores = scores + jnp.maximum(lg, 0.0) * w[:, h:h + 1]
    t_ids = i * blk + lax.broadcasted_iota(jnp.int32, (blk, S), 0)
    s_ids = lax.broadcasted_iota(jnp.int32, (blk, S), 1)
    sentinel = jnp.float32(-1e9) - 64.0 * s_ids.astype(jnp.float32)
    scores = jnp.where(s_ids > t_ids, sentinel, scores)
    v, ix = _bitonic_desc(scores, s_ids, S)
    vals = v[:, :topk]
    vals = jnp.where(vals < -1e8, jnp.float32(-1e9), vals)
    vals_ref[0] = vals
    idx_ref[0] = ix[:, :topk]


def kernel(x, wq, wk, ww):
    B, S, D = x.shape
    DH = wk.shape[1]
    H = ww.shape[1]
    half = DH // 2
    topk = S // 2
    blk = min(256, S)
    nblk = S // blk

    perm = np.concatenate([np.arange(0, DH, 2), np.arange(1, DH, 2)])
    permq = np.concatenate([h * DH + perm for h in range(H)])
    wq_p = wq[:, permq]
    wk_p = wk[:, perm]
    cos_np, sin_np = _rope_tables(S, DH)
    cos = jnp.asarray(cos_np)
    sin = jnp.asarray(sin_np)

    krot = pl.pallas_call(
        _k_body,
        grid=(B, nblk),
        in_specs=[
            pl.BlockSpec((1, blk, D), lambda b, i: (b, i, 0)),
            pl.BlockSpec((D, DH), lambda b, i: (0, 0)),
            pl.BlockSpec((blk, half), lambda b, i: (i, 0)),
            pl.BlockSpec((blk, half), lambda b, i: (i, 0)),
        ],
        out_specs=pl.BlockSpec((1, blk, DH), lambda b, i: (b, i, 0)),
        out_shape=jax.ShapeDtypeStruct((B, S, DH), jnp.float32),
    )(x, wk_p, cos, sin)

    body = functools.partial(_score_body, H=H, DH=DH, blk=blk, topk=topk, S=S)
    vals, idx = pl.pallas_call(
        body,
        grid=(B, nblk),
        in_specs=[
            pl.BlockSpec((1, blk, D), lambda b, i: (b, i, 0)),
            pl.BlockSpec((D, H * DH), lambda b, i: (0, 0)),
            pl.BlockSpec((D, H), lambda b, i: (0, 0)),
            pl.BlockSpec((1, S, DH), lambda b, i: (b, 0, 0)),
            pl.BlockSpec((blk, half), lambda b, i: (i, 0)),
            pl.BlockSpec((blk, half), lambda b, i: (i, 0)),
        ],
        out_specs=[
            pl.BlockSpec((1, blk, topk), lambda b, i: (b, i, 0)),
            pl.BlockSpec((1, blk, topk), lambda b, i: (b, i, 0)),
        ],
        out_shape=[
            jax.ShapeDtypeStruct((B, S, topk), jnp.float32),
            jax.ShapeDtypeStruct((B, S, topk), jnp.int32),
        ],
    )(x, wq_p, ww, krot, cos, sin)
    return vals, idx
